# baseline (speedup 1.0000x reference)
import functools

import jax
import jax.numpy as jnp
from jax import lax
from jax.experimental import pallas as pl
from jax.experimental.pallas import tpu as pltpu
from jax.experimental.pallas import tpu_sc as plsc

VOCAB = 100000
VIDEO_DIM = 512
EMBED = 512
BATCH = 16384
HIST = 50
HIST_PAD = 56

NC = 2
NS = 16
NW = NC * NS
BAGS_PER_W = BATCH // NW
HALF = BAGS_PER_W // 2
LANES = 16
CHUNKS = EMBED // LANES
GROUP = 8
NBUF = 3


def _sc_bag_body(text_hbm, table_hbm, out_hbm,
                 idx_v, rows0, rows1, rows2, stage_v,
                 sem0, sem1, sem2, sem_out):
    wid = lax.axis_index("s") * NC + lax.axis_index("c")
    base = wid * BAGS_PER_W
    rows = (rows0, rows1, rows2)
    sems = (sem0, sem1, sem2)
    inv = jnp.float32(1.0 / HIST)

    for half in range(2):
        hbase = base + half * HALF
        pltpu.sync_copy(text_hbm.at[pl.ds(hbase, HALF)], idx_v)

        for b in range(NBUF):
            pltpu.async_copy(table_hbm.at[idx_v.at[b]], rows[b], sems[b])

        def tri_body(i, carry):
            for b in range(NBUF):
                j = NBUF * i + b
                r = rows[b]
                g = j // GROUP
                s = lax.rem(g, 2)
                row = lax.rem(j, GROUP)

                @pl.when(jnp.logical_and(row == 0, g >= 2))
                def _():
                    pltpu.make_async_copy(
                        stage_v.at[0], out_hbm.at[pl.ds(hbase, GROUP)],
                        sem_out).wait()

                pltpu.make_async_copy(table_hbm.at[idx_v.at[j]], r,
                                      sems[b]).wait()

                def accum(rr, accs):
                    return tuple(accs[c] + r[rr, pl.ds(c * LANES, LANES)]
                                 for c in range(CHUNKS))

                accs = lax.fori_loop(
                    0, HIST, accum,
                    tuple(jnp.zeros((LANES,), jnp.float32)
                          for _ in range(CHUNKS)))

                @pl.when(j + NBUF < HALF)
                def _():
                    pltpu.async_copy(table_hbm.at[idx_v.at[j + NBUF]], r,
                                     sems[b])

                for c in range(CHUNKS):
                    stage_v[s, row, pl.ds(c * LANES, LANES)] = accs[c] * inv

                @pl.when(row == GROUP - 1)
                def _():
                    pltpu.async_copy(
                        stage_v.at[s],
                        out_hbm.at[pl.ds(hbase + g * GROUP, GROUP)],
                        sem_out)
            return carry

        lax.fori_loop(0, HALF // NBUF, tri_body, 0)

        rem = HALF - (HALF // NBUF) * NBUF
        for b in range(rem):
            j = HALF - rem + b
            r = rows[b]
            pltpu.make_async_copy(table_hbm.at[idx_v.at[j]], r,
                                  sems[b]).wait()

            def accum(rr, accs):
                return tuple(accs[c] + r[rr, pl.ds(c * LANES, LANES)]
                             for c in range(CHUNKS))

            accs = lax.fori_loop(
                0, HIST, accum,
                tuple(jnp.zeros((LANES,), jnp.float32)
                      for _ in range(CHUNKS)))
            row = (HALF - rem + b) % GROUP
            s = ((HALF - rem + b) // GROUP) % 2
            for c in range(CHUNKS):
                stage_v[s, row, pl.ds(c * LANES, LANES)] = accs[c] * inv
            if row == GROUP - 1:
                pltpu.async_copy(
                    stage_v.at[s],
                    out_hbm.at[pl.ds(hbase + (j // GROUP) * GROUP, GROUP)],
                    sem_out)

        for _ in range(2):
            pltpu.make_async_copy(stage_v.at[0],
                                  out_hbm.at[pl.ds(hbase, GROUP)],
                                  sem_out).wait()


_sc_bag = functools.partial(
    pl.kernel,
    out_type=jax.ShapeDtypeStruct((BATCH, EMBED), jnp.float32),
    mesh=plsc.VectorSubcoreMesh(core_axis_name="c", subcore_axis_name="s"),
    compiler_params=pltpu.CompilerParams(use_tc_tiling_on_sc=False),
    scratch_types=[
        pltpu.VMEM((HALF, HIST_PAD), jnp.int32),
        pltpu.VMEM((HIST_PAD, EMBED), jnp.float32),
        pltpu.VMEM((HIST_PAD, EMBED), jnp.float32),
        pltpu.VMEM((HIST_PAD, EMBED), jnp.float32),
        pltpu.VMEM((2, GROUP, EMBED), jnp.float32),
        pltpu.SemaphoreType.DMA,
        pltpu.SemaphoreType.DMA,
        pltpu.SemaphoreType.DMA,
        pltpu.SemaphoreType.DMA,
    ],
)(_sc_bag_body)


BM = 1024


def _tc_body(video_ref, w_ref, b_ref, t_ref, out_ref):
    mm = lax.dot_general(video_ref[...], w_ref[...],
                         (((1,), (1,)), ((), ())),
                         preferred_element_type=jnp.float32)
    out_ref[:, :EMBED] = mm + b_ref[...]
    out_ref[:, EMBED:] = t_ref[...]


def kernel(video, text, W, b, table):
    text_pad = jnp.pad(text.astype(jnp.int32), ((0, 0), (0, HIST_PAD - HIST)))
    text_embed = _sc_bag(text_pad, table)
    out = pl.pallas_call(
        _tc_body,
        grid=(BATCH // BM,),
        in_specs=[
            pl.BlockSpec((BM, VIDEO_DIM), lambda i: (i, 0)),
            pl.BlockSpec((EMBED, VIDEO_DIM), lambda i: (0, 0)),
            pl.BlockSpec((1, EMBED), lambda i: (0, 0)),
            pl.BlockSpec((BM, EMBED), lambda i: (i, 0)),
        ],
        out_specs=pl.BlockSpec((BM, 2 * EMBED), lambda i: (i, 0)),
        out_shape=jax.ShapeDtypeStruct((BATCH, 2 * EMBED), jnp.float32),
    )(video, W, b.reshape(1, EMBED), text_embed)
    return out


# speedup vs baseline: 1.0174x; 1.0174x over previous
import functools

import jax
import jax.numpy as jnp
from jax import lax
from jax.experimental import pallas as pl
from jax.experimental.pallas import tpu as pltpu
from jax.experimental.pallas import tpu_sc as plsc

VOCAB = 100000
VIDEO_DIM = 512
EMBED = 512
BATCH = 16384
HIST = 50
HIST_PAD = 56

NC = 2
NS = 16
NW = NC * NS
BAGS_PER_W = BATCH // NW
LANES = 16
PCHUNKS = EMBED // (2 * LANES)
GROUP = 8
NBUF = 4


def _sc_bag_body(text_hbm, table_hbm, out_hbm,
                 idx_v, rows0, rows1, rows2, rows3, stage_v,
                 sem0, sem1, sem2, sem3, sem_out):
    wid = lax.axis_index("s") * NC + lax.axis_index("c")
    base = wid * BAGS_PER_W
    rows = (rows0, rows1, rows2, rows3)
    sems = (sem0, sem1, sem2, sem3)
    inv = jnp.float32(1.0 / HIST)
    lane2 = lax.iota(jnp.int32, LANES) * 2
    hi_mask = jnp.int32(-65536)

    pltpu.sync_copy(text_hbm.at[pl.ds(base, BAGS_PER_W)], idx_v)

    def fire_bag(j, b):
        pltpu.async_copy(table_hbm.at[idx_v.at[j]], rows[b], sems[b])

    for b in range(NBUF):
        fire_bag(b, b)

    def quad_body(i, carry):
        for b in range(NBUF):
            j = NBUF * i + b
            r = rows[b]
            g = j // GROUP
            s = lax.rem(g, 2)
            row = lax.rem(j, GROUP)

            @pl.when(jnp.logical_and(row == 0, g >= 2))
            def _():
                pltpu.make_async_copy(
                    stage_v.at[0], out_hbm.at[pl.ds(base, GROUP)],
                    sem_out).wait()

            pltpu.make_async_copy(table_hbm.at[pl.ds(0, HIST_PAD)], r,
                                  sems[b]).wait()

            def accum(rr, accs):
                out = []
                for c in range(PCHUNKS):
                    packed = plsc.bitcast(
                        r[rr, pl.ds(c * 2 * LANES, 2 * LANES)], jnp.int32)
                    lo = plsc.bitcast(lax.shift_left(packed, 16),
                                      jnp.float32)
                    hi = plsc.bitcast(lax.bitwise_and(packed, hi_mask),
                                      jnp.float32)
                    out.append(accs[2 * c] + lo)
                    out.append(accs[2 * c + 1] + hi)
                return tuple(out)

            accs = lax.fori_loop(
                0, HIST, accum,
                tuple(jnp.zeros((LANES,), jnp.float32)
                      for _ in range(2 * PCHUNKS)))

            @pl.when(j + NBUF < BAGS_PER_W)
            def _():
                fire_bag(j + NBUF, b)

            s_vec = jnp.full((LANES,), s, jnp.int32)
            row_vec = jnp.full((LANES,), row, jnp.int32)
            for c in range(PCHUNKS):
                pos = lane2 + (c * 2 * LANES)
                plsc.store_scatter(stage_v, [s_vec, row_vec, pos],
                                   accs[2 * c] * inv)
                plsc.store_scatter(stage_v, [s_vec, row_vec, pos + 1],
                                   accs[2 * c + 1] * inv)

            @pl.when(row == GROUP - 1)
            def _():
                pltpu.async_copy(
                    stage_v.at[s],
                    out_hbm.at[pl.ds(base + g * GROUP, GROUP)],
                    sem_out)
        return carry

    lax.fori_loop(0, BAGS_PER_W // NBUF, quad_body, 0)

    for _ in range(2):
        pltpu.make_async_copy(stage_v.at[0],
                              out_hbm.at[pl.ds(base, GROUP)],
                              sem_out).wait()


_sc_bag = functools.partial(
    pl.kernel,
    out_type=jax.ShapeDtypeStruct((BATCH, EMBED), jnp.float32),
    mesh=plsc.VectorSubcoreMesh(core_axis_name="c", subcore_axis_name="s"),
    compiler_params=pltpu.CompilerParams(use_tc_tiling_on_sc=False,
                                         needs_layout_passes=False),
    scratch_types=[
        pltpu.VMEM((BAGS_PER_W, HIST_PAD), jnp.int32),
        pltpu.VMEM((HIST_PAD, EMBED), jnp.bfloat16),
        pltpu.VMEM((HIST_PAD, EMBED), jnp.bfloat16),
        pltpu.VMEM((HIST_PAD, EMBED), jnp.bfloat16),
        pltpu.VMEM((HIST_PAD, EMBED), jnp.bfloat16),
        pltpu.VMEM((2, GROUP, EMBED), jnp.float32),
        pltpu.SemaphoreType.DMA,
        pltpu.SemaphoreType.DMA,
        pltpu.SemaphoreType.DMA,
        pltpu.SemaphoreType.DMA,
        pltpu.SemaphoreType.DMA,
    ],
)(_sc_bag_body)


BM = 1024


def _tc_body(video_ref, w_ref, b_ref, t_ref, out_ref):
    mm = lax.dot_general(video_ref[...], w_ref[...],
                         (((1,), (1,)), ((), ())),
                         preferred_element_type=jnp.float32)
    out_ref[:, :EMBED] = mm + b_ref[...]
    out_ref[:, EMBED:] = t_ref[...]


def kernel(video, text, W, b, table):
    text_pad = jnp.pad(text.astype(jnp.int32), ((0, 0), (0, HIST_PAD - HIST)))
    text_embed = _sc_bag(text_pad, table.astype(jnp.bfloat16))
    out = pl.pallas_call(
        _tc_body,
        grid=(BATCH // BM,),
        in_specs=[
            pl.BlockSpec((BM, VIDEO_DIM), lambda i: (i, 0)),
            pl.BlockSpec((EMBED, VIDEO_DIM), lambda i: (0, 0)),
            pl.BlockSpec((1, EMBED), lambda i: (0, 0)),
            pl.BlockSpec((BM, EMBED), lambda i: (i, 0)),
        ],
        out_specs=pl.BlockSpec((BM, 2 * EMBED), lambda i: (i, 0)),
        out_shape=jax.ShapeDtypeStruct((BATCH, 2 * EMBED), jnp.float32),
    )(video, W, b.reshape(1, EMBED), text_embed)
    return out


# speedup vs baseline: 5.4700x; 5.3767x over previous
import functools

import jax
import jax.numpy as jnp
from jax import lax
from jax.experimental import pallas as pl
from jax.experimental.pallas import tpu as pltpu
from jax.experimental.pallas import tpu_sc as plsc

VOCAB = 100000
VIDEO_DIM = 512
EMBED = 512
BATCH = 16384
HIST = 50

NC = 2
NS = 16
NW = NC * NS
BAGS_PER_W = BATCH // NW
HALF = BAGS_PER_W // 2
LANES = 16
CHUNKS = EMBED // LANES
GROUP = 8
NBUF = 4


def _sc_bag_body(text_hbm, table_hbm, out_hbm,
                 idx_v, rows0, rows1, rows2, rows3, stage_v,
                 sem0, sem1, sem2, sem3, sem_out):
    wid = lax.axis_index("s") * NC + lax.axis_index("c")
    base = wid * BAGS_PER_W
    rows = (rows0, rows1, rows2, rows3)
    sems = (sem0, sem1, sem2, sem3)
    inv = jnp.float32(1.0 / HIST)
    lane_iota = lax.iota(jnp.int32, LANES)

    def fire_bag(j, b):
        vecs = [idx_v[j, pl.ds(t * LANES, LANES)] for t in range(3)]
        tail = idx_v[j, pl.ds(HIST - LANES, LANES)]
        for k in range(HIST):
            if k < 48:
                vec, lane = vecs[k // LANES], k % LANES
            else:
                vec, lane = tail, k - (HIST - LANES)
            i = lax.reduce_sum(
                jnp.where(lane_iota == lane, vec, 0), axes=(0,))
            pltpu.async_copy(table_hbm.at[i], rows[b].at[k], sems[b])

    for half in range(2):
        hbase = base + half * HALF
        pltpu.sync_copy(text_hbm.at[pl.ds(hbase, HALF)], idx_v)
        for b in range(NBUF):
            fire_bag(b, b)

        def quad_body(i, carry):
            for b in range(NBUF):
                j = NBUF * i + b
                r = rows[b]
                g = j // GROUP
                s = lax.rem(g, 2)
                row = lax.rem(j, GROUP)

                @pl.when(jnp.logical_and(row == 0, g >= 2))
                def _():
                    pltpu.make_async_copy(
                        stage_v.at[0], out_hbm.at[pl.ds(hbase, GROUP)],
                        sem_out).wait()

                pltpu.make_async_copy(table_hbm.at[pl.ds(0, HIST)], r,
                                      sems[b]).wait()

                def accum(rr, accs):
                    return tuple(accs[c] + r[rr, pl.ds(c * LANES, LANES)]
                                 for c in range(CHUNKS))

                accs = lax.fori_loop(
                    0, HIST, accum,
                    tuple(jnp.zeros((LANES,), jnp.float32)
                          for _ in range(CHUNKS)))

                @pl.when(j + NBUF < HALF)
                def _():
                    fire_bag(j + NBUF, b)

                for c in range(CHUNKS):
                    stage_v[s, row, pl.ds(c * LANES, LANES)] = accs[c] * inv

                @pl.when(row == GROUP - 1)
                def _():
                    pltpu.async_copy(
                        stage_v.at[s],
                        out_hbm.at[pl.ds(hbase + g * GROUP, GROUP)],
                        sem_out)
            return carry

        lax.fori_loop(0, HALF // NBUF, quad_body, 0)

        for _ in range(2):
            pltpu.make_async_copy(stage_v.at[0],
                                  out_hbm.at[pl.ds(hbase, GROUP)],
                                  sem_out).wait()


_sc_bag = functools.partial(
    pl.kernel,
    out_type=jax.ShapeDtypeStruct((BATCH, EMBED), jnp.float32),
    mesh=plsc.VectorSubcoreMesh(core_axis_name="c", subcore_axis_name="s"),
    compiler_params=pltpu.CompilerParams(use_tc_tiling_on_sc=False,
                                         needs_layout_passes=False),
    scratch_types=[
        pltpu.VMEM((HALF, HIST), jnp.int32),
        pltpu.VMEM((HIST, EMBED), jnp.float32),
        pltpu.VMEM((HIST, EMBED), jnp.float32),
        pltpu.VMEM((HIST, EMBED), jnp.float32),
        pltpu.VMEM((HIST, EMBED), jnp.float32),
        pltpu.VMEM((2, GROUP, EMBED), jnp.float32),
        pltpu.SemaphoreType.DMA,
        pltpu.SemaphoreType.DMA,
        pltpu.SemaphoreType.DMA,
        pltpu.SemaphoreType.DMA,
        pltpu.SemaphoreType.DMA,
    ],
)(_sc_bag_body)


BM = 1024


def _tc_body(video_ref, w_ref, b_ref, t_ref, out_ref):
    mm = lax.dot_general(video_ref[...], w_ref[...],
                         (((1,), (1,)), ((), ())),
                         preferred_element_type=jnp.float32)
    out_ref[:, :EMBED] = mm + b_ref[...]
    out_ref[:, EMBED:] = t_ref[...]


def kernel(video, text, W, b, table):
    text_embed = _sc_bag(text.astype(jnp.int32), table)
    out = pl.pallas_call(
        _tc_body,
        grid=(BATCH // BM,),
        in_specs=[
            pl.BlockSpec((BM, VIDEO_DIM), lambda i: (i, 0)),
            pl.BlockSpec((EMBED, VIDEO_DIM), lambda i: (0, 0)),
            pl.BlockSpec((1, EMBED), lambda i: (0, 0)),
            pl.BlockSpec((BM, EMBED), lambda i: (i, 0)),
        ],
        out_specs=pl.BlockSpec((BM, 2 * EMBED), lambda i: (i, 0)),
        out_shape=jax.ShapeDtypeStruct((BATCH, 2 * EMBED), jnp.float32),
    )(video, W, b.reshape(1, EMBED), text_embed)
    return out


# speedup vs baseline: 5.6799x; 1.0384x over previous
import functools

import jax
import jax.numpy as jnp
from jax import lax
from jax.experimental import pallas as pl
from jax.experimental.pallas import tpu as pltpu
from jax.experimental.pallas import tpu_sc as plsc

VOCAB = 100000
VIDEO_DIM = 512
EMBED = 512
BATCH = 16384
HIST = 50

NC = 2
NS = 16
NW = NC * NS
BAGS_PER_W = BATCH // NW
HALF = BAGS_PER_W // 2
LANES = 16
CHUNKS = EMBED // LANES
GROUP = 8
NBUF = 4


def _sc_bag_body(text_hbm, table_hbm, out_hbm,
                 idx_v, rows0, rows1, rows2, rows3, stage_v,
                 sem0, sem1, sem2, sem3, sem_out):
    wid = lax.axis_index("s") * NC + lax.axis_index("c")
    base = wid * BAGS_PER_W
    rows = (rows0, rows1, rows2, rows3)
    sems = (sem0, sem1, sem2, sem3)
    inv = jnp.float32(1.0 / HIST)
    lane_iota = lax.iota(jnp.int32, LANES)

    def fire_bag(j, b):
        vecs = [idx_v[j, pl.ds(t * LANES, LANES)] for t in range(3)]
        tail = idx_v[j, pl.ds(HIST - LANES, LANES)]
        for k in range(HIST):
            if k < 48:
                vec, lane = vecs[k // LANES], k % LANES
            else:
                vec, lane = tail, k - (HIST - LANES)
            i = lax.reduce_sum(
                jnp.where(lane_iota == lane, vec, 0), axes=(0,))
            pltpu.async_copy(table_hbm.at[i], rows[b].at[k], sems[b])

    for half in range(2):
        hbase = base + half * HALF
        pltpu.sync_copy(text_hbm.at[pl.ds(hbase, HALF)], idx_v)
        for b in range(NBUF):
            fire_bag(b, b)

        def quad_body(i, carry):
            for b in range(NBUF):
                j = NBUF * i + b
                r = rows[b]
                g = j // GROUP
                s = lax.rem(g, 2)
                row = lax.rem(j, GROUP)

                @pl.when(jnp.logical_and(row == 0, g >= 2))
                def _():
                    pltpu.make_async_copy(
                        stage_v.at[0], out_hbm.at[pl.ds(hbase, GROUP)],
                        sem_out).wait()

                pltpu.make_async_copy(table_hbm.at[pl.ds(0, HIST)], r,
                                      sems[b]).wait()

                accs = tuple(r[0, pl.ds(c * LANES, LANES)]
                             for c in range(CHUNKS))

                @pl.when(j + NBUF < HALF)
                def _():
                    fire_bag(j + NBUF, b)

                for c in range(CHUNKS):
                    stage_v[s, row, pl.ds(c * LANES, LANES)] = accs[c] * inv

                @pl.when(row == GROUP - 1)
                def _():
                    pltpu.async_copy(
                        stage_v.at[s],
                        out_hbm.at[pl.ds(hbase + g * GROUP, GROUP)],
                        sem_out)
            return carry

        lax.fori_loop(0, HALF // NBUF, quad_body, 0)

        for _ in range(2):
            pltpu.make_async_copy(stage_v.at[0],
                                  out_hbm.at[pl.ds(hbase, GROUP)],
                                  sem_out).wait()


_sc_bag = functools.partial(
    pl.kernel,
    out_type=jax.ShapeDtypeStruct((BATCH, EMBED), jnp.float32),
    mesh=plsc.VectorSubcoreMesh(core_axis_name="c", subcore_axis_name="s"),
    compiler_params=pltpu.CompilerParams(use_tc_tiling_on_sc=False,
                                         needs_layout_passes=False),
    scratch_types=[
        pltpu.VMEM((HALF, HIST), jnp.int32),
        pltpu.VMEM((HIST, EMBED), jnp.float32),
        pltpu.VMEM((HIST, EMBED), jnp.float32),
        pltpu.VMEM((HIST, EMBED), jnp.float32),
        pltpu.VMEM((HIST, EMBED), jnp.float32),
        pltpu.VMEM((2, GROUP, EMBED), jnp.float32),
        pltpu.SemaphoreType.DMA,
        pltpu.SemaphoreType.DMA,
        pltpu.SemaphoreType.DMA,
        pltpu.SemaphoreType.DMA,
        pltpu.SemaphoreType.DMA,
    ],
)(_sc_bag_body)


BM = 1024


def _tc_body(video_ref, w_ref, b_ref, t_ref, out_ref):
    mm = lax.dot_general(video_ref[...], w_ref[...],
                         (((1,), (1,)), ((), ())),
                         preferred_element_type=jnp.float32)
    out_ref[:, :EMBED] = mm + b_ref[...]
    out_ref[:, EMBED:] = t_ref[...]


def kernel(video, text, W, b, table):
    text_embed = _sc_bag(text.astype(jnp.int32), table)
    out = pl.pallas_call(
        _tc_body,
        grid=(BATCH // BM,),
        in_specs=[
            pl.BlockSpec((BM, VIDEO_DIM), lambda i: (i, 0)),
            pl.BlockSpec((EMBED, VIDEO_DIM), lambda i: (0, 0)),
            pl.BlockSpec((1, EMBED), lambda i: (0, 0)),
            pl.BlockSpec((BM, EMBED), lambda i: (i, 0)),
        ],
        out_specs=pl.BlockSpec((BM, 2 * EMBED), lambda i: (i, 0)),
        out_shape=jax.ShapeDtypeStruct((BATCH, 2 * EMBED), jnp.float32),
    )(video, W, b.reshape(1, EMBED), text_embed)
    return out


# speedup vs baseline: 5.6860x; 1.0011x over previous
import functools

import jax
import jax.numpy as jnp
from jax import lax
from jax.experimental import pallas as pl
from jax.experimental.pallas import tpu as pltpu
from jax.experimental.pallas import tpu_sc as plsc

VOCAB = 100000
VIDEO_DIM = 512
EMBED = 512
BATCH = 16384
HIST = 50

NC = 2
NS = 16
NW = NC * NS
BAGS_PER_W = BATCH // NW
HALF = BAGS_PER_W // 2
LANES = 16
CHUNKS = EMBED // LANES
GROUP = 8
NBUF = 4


def _sc_bag_body(text_hbm, table_hbm, out_hbm,
                 idx_v, rows0, rows1, rows2, rows3, stage_v,
                 sem0, sem1, sem2, sem3, sem_out):
    wid = lax.axis_index("s") * NC + lax.axis_index("c")
    base = wid * BAGS_PER_W
    rows = (rows0, rows1, rows2, rows3)
    sems = (sem0, sem1, sem2, sem3)
    inv = jnp.float32(1.0 / HIST)
    lane_iota = lax.iota(jnp.int32, LANES)

    def fire_bag(j, b):
        vecs = [idx_v[j, pl.ds(t * LANES, LANES)] for t in range(3)]
        tail = idx_v[j, pl.ds(HIST - LANES, LANES)]
        for k in range(HIST):
            if k < 48:
                vec, lane = vecs[k // LANES], k % LANES
            else:
                vec, lane = tail, k - (HIST - LANES)
            i = lax.reduce_sum(
                jnp.where(lane_iota == lane, vec, 0), axes=(0,))
            pltpu.async_copy(table_hbm.at[i], rows[b].at[k], sems[b])

    for half in range(2):
        hbase = base + half * HALF
        pltpu.sync_copy(text_hbm.at[pl.ds(hbase, HALF)], idx_v)
        for b in range(NBUF):
            fire_bag(b, b)

        def quad_body(i, carry):
            for b in range(NBUF):
                j = NBUF * i + b
                r = rows[b]
                g = j // GROUP
                s = lax.rem(g, 2)
                row = lax.rem(j, GROUP)

                @pl.when(jnp.logical_and(row == 0, g >= 2))
                def _():
                    pltpu.make_async_copy(
                        stage_v.at[0], out_hbm.at[pl.ds(hbase, GROUP)],
                        sem_out).wait()

                pltpu.make_async_copy(table_hbm.at[pl.ds(0, HIST)], r,
                                      sems[b]).wait()

                def accum(rr, accs):
                    return tuple(accs[c] + r[rr, pl.ds(c * LANES, LANES)]
                                 for c in range(CHUNKS))

                accs = lax.fori_loop(
                    0, HIST, accum,
                    tuple(jnp.zeros((LANES,), jnp.float32)
                          for _ in range(CHUNKS)))

                @pl.when(j + NBUF < HALF)
                def _():
                    fire_bag(j + NBUF, b)

                for c in range(CHUNKS):
                    stage_v[s, row, pl.ds(c * LANES, LANES)] = accs[c] * inv

                @pl.when(row == GROUP - 1)
                def _():
                    pltpu.async_copy(
                        stage_v.at[s],
                        out_hbm.at[pl.ds(hbase + g * GROUP, GROUP)],
                        sem_out)
            return carry

        lax.fori_loop(0, HALF // NBUF, quad_body, 0)

        for _ in range(2):
            pltpu.make_async_copy(stage_v.at[0],
                                  out_hbm.at[pl.ds(hbase, GROUP)],
                                  sem_out).wait()


_sc_bag = functools.partial(
    pl.kernel,
    out_type=jax.ShapeDtypeStruct((BATCH, EMBED), jnp.float32),
    mesh=plsc.VectorSubcoreMesh(core_axis_name="c", subcore_axis_name="s"),
    compiler_params=pltpu.CompilerParams(use_tc_tiling_on_sc=False,
                                         needs_layout_passes=False),
    scratch_types=[
        pltpu.VMEM((HALF, HIST), jnp.int32),
        pltpu.VMEM((HIST, EMBED), jnp.float32),
        pltpu.VMEM((HIST, EMBED), jnp.float32),
        pltpu.VMEM((HIST, EMBED), jnp.float32),
        pltpu.VMEM((HIST, EMBED), jnp.float32),
        pltpu.VMEM((2, GROUP, EMBED), jnp.float32),
        pltpu.SemaphoreType.DMA,
        pltpu.SemaphoreType.DMA,
        pltpu.SemaphoreType.DMA,
        pltpu.SemaphoreType.DMA,
        pltpu.SemaphoreType.DMA,
    ],
)(_sc_bag_body)


BM = 1024


def _tc_body(video_ref, w_ref, b_ref, t_ref, out_ref):
    mm = lax.dot_general(video_ref[...], w_ref[...],
                         (((1,), (1,)), ((), ())),
                         preferred_element_type=jnp.float32)
    out_ref[:, :EMBED] = mm + b_ref[...]
    out_ref[:, EMBED:] = t_ref[...].reshape(BM, EMBED)


def kernel(video, text, W, b, table):
    text_embed = _sc_bag(text.astype(jnp.int32), table)
    out = pl.pallas_call(
        _tc_body,
        grid=(BATCH // BM,),
        in_specs=[
            pl.BlockSpec((BM, VIDEO_DIM), lambda i: (i, 0)),
            pl.BlockSpec((EMBED, VIDEO_DIM), lambda i: (0, 0)),
            pl.BlockSpec((1, EMBED), lambda i: (0, 0)),
            pl.BlockSpec((BM * EMBED,), lambda i: (i,)),
        ],
        out_specs=pl.BlockSpec((BM, 2 * EMBED), lambda i: (i, 0)),
        out_shape=jax.ShapeDtypeStruct((BATCH, 2 * EMBED), jnp.float32),
    )(video, W, b.reshape(1, EMBED), text_embed.reshape(BATCH * EMBED))
    return out
